# Initial kernel scaffold; baseline (speedup 1.0000x reference)
import jax
import jax.numpy as jnp
from jax.experimental import pallas as pl


def kernel(points, params):
    raise NotImplementedError("write your pallas kernel here")


# speedup vs baseline: 54.9215x; 54.9215x over previous
import functools
import math

import jax
import jax.numpy as jnp
from jax import lax
from jax.experimental import pallas as pl
from jax.experimental.pallas import tpu as pltpu

N = 512
H = 512
K = 16
ND = 3
NEG = -1e30
BIG = 3.4e38

F32 = jnp.float32


def _dot(a, b):
    return jax.lax.dot_general(a, b, (((1,), (0,)), ((), ())),
                               preferred_element_type=F32)


def _dot_t(a, b):
    return jax.lax.dot_general(a, b, (((1,), (1,)), ((), ())),
                               preferred_element_type=F32)


def _layernorm(x, scale, bias, eps=1e-6):
    mu = jnp.mean(x, axis=-1, keepdims=True)
    var = jnp.mean(jnp.square(x - mu), axis=-1, keepdims=True)
    return (x - mu) * jax.lax.rsqrt(var + eps) * scale + bias


def _topk_mask(D):
    iota = jax.lax.broadcasted_iota(jnp.int32, (N, N), 1)
    M = jnp.zeros((N, N), jnp.bool_)
    Dw = D
    for _ in range(K):
        rmin = jnp.min(Dw, axis=1, keepdims=True)
        eq = Dw == rmin
        fidx = jnp.min(jnp.where(eq, iota, N), axis=1, keepdims=True)
        hit = iota == fidx
        M = jnp.logical_or(M, hit)
        Dw = jnp.where(hit, BIG, Dw)
    return M


def _pt_layer(x, pos, posT, M, Wqkv, bqkv, Wpe, bpe, Wpd, bpd, wa, ba,
              Wo, bo, lns, lnb):
    qkv = _dot(x, Wqkv) + bqkv
    q = qkv[:, :H]
    k = qkv[:, H:2 * H]
    v = qkv[:, 2 * H:]
    pe = _dot(pos, Wpe) + bpe
    qq = q + pe
    qw = qq * wa
    kpe = k + pe
    u = _dot_t(qw, Wpd)
    c = (jnp.sum(qw * bpd, axis=1, keepdims=True) + ba
         - jnp.sum(u * pos, axis=1, keepdims=True))
    L = _dot_t(qw, kpe) + _dot_t(u, pos) + c
    Lm = jnp.where(M, L, NEG)
    rmax = jnp.max(Lm, axis=1, keepdims=True)
    e = jnp.where(M, jnp.exp(Lm - rmax), 0.0)
    A = e / jnp.sum(e, axis=1, keepdims=True)
    out = _dot(A, v)
    y = jax.nn.relu(_dot(out, Wo) + bo)
    x = x + y
    return _layernorm(x, lns, lnb)


def _cloud_kernel(pts_ref, pos_ref, posT_ref, *rest):
    (W0, b0,
     Wqkv0, bqkv0, Wpe0, bpe0, Wpd0, bpd0, wa0, ba0, Wo0, bo0, lns0, lnb0,
     Wqkv1, bqkv1, Wpe1, bpe1, Wpd1, bpd1, wa1, ba1, Wo1, bo1, lns1, lnb1,
     out_ref) = rest
    pts = pts_ref[0]
    pos = pos_ref[0]
    posT = posT_ref[0]

    D = jnp.zeros((N, N), F32)
    for d in range(ND):
        pn = pos[:, d:d + 1]
        pm = posT[d:d + 1, :]
        diff = pn - pm
        D = D + diff * diff
    M = _topk_mask(D)

    x = _dot(pts, W0[...]) + b0[...]
    x = _pt_layer(x, pos, posT, M, Wqkv0[...], bqkv0[...], Wpe0[...],
                  bpe0[...], Wpd0[...], bpd0[...], wa0[...], ba0[...],
                  Wo0[...], bo0[...], lns0[...], lnb0[...])
    x = _pt_layer(x, pos, posT, M, Wqkv1[...], bqkv1[...], Wpe1[...],
                  bpe1[...], Wpd1[...], bpd1[...], wa1[...], ba1[...],
                  Wo1[...], bo1[...], lns1[...], lnb1[...])
    out_ref[0, 0] = jnp.max(x, axis=0)


def _enc_kernel(x_ref, *rest):
    (Wk, bk, Wq, bq, Wv, bv, Wo1, bo1, Wo2, bo2, lns, lnb, out_ref) = rest
    Bn = x_ref.shape[0]
    scale = 1.0 / math.sqrt(float(H))
    for b in range(Bn):
        xb = x_ref[b]
        k = _dot(xb, Wk[...]) + bk[...]
        q = _dot(xb, Wq[...]) + bq[...]
        v = _dot(xb, Wv[...]) + bv[...]
        attn = _dot_t(q, k) * scale
        attn = attn - jnp.max(attn, axis=1, keepdims=True)
        e = jnp.exp(attn)
        attn = e / jnp.sum(e, axis=1, keepdims=True)
        out = _dot(attn, v)
        out = jax.nn.relu(_dot(out, Wo1[...]) + bo1[...])
        out = _dot(out, Wo2[...]) + bo2[...]
        xo = _layernorm(xb + out, lns[...], lnb[...])
        out_ref[b] = jnp.max(xo, axis=0)


def _row(a):
    return a.reshape(1, -1)


@jax.jit
def kernel(points, params):
    B, S, Np, C = points.shape
    G = B * S
    pts = points.reshape(G, Np, C)
    pos = pts[..., :ND]
    posT = jnp.swapaxes(pos, 1, 2)

    p = params
    args = [p['W0'], _row(p['b0'])]
    for i in range(2):
        lp = p['layer%d' % i]
        args += [
            jnp.concatenate([lp['Wq'], lp['Wk'], lp['Wv']], axis=1),
            _row(jnp.concatenate([lp['bq'], lp['bk'], lp['bv']])),
            lp['Wpe'], _row(lp['bpe']),
            lp['Wpd'], _row(lp['bpd']),
            lp['Wa'].reshape(1, H), lp['ba'].reshape(1, 1),
            lp['Wo'], _row(lp['bo']),
            _row(lp['ln_scale']), _row(lp['ln_bias']),
        ]

    rep = [pl.BlockSpec(a.shape, lambda g, nd=a.ndim: (0,) * nd) for a in args]
    grid = (G,)
    xp = pl.pallas_call(
        _cloud_kernel,
        grid=grid,
        in_specs=[
            pl.BlockSpec((1, Np, C), lambda g: (g, 0, 0)),
            pl.BlockSpec((1, Np, ND), lambda g: (g, 0, 0)),
            pl.BlockSpec((1, ND, Np), lambda g: (g, 0, 0)),
        ] + rep,
        out_specs=pl.BlockSpec((1, 1, H), lambda g: (g, 0, 0)),
        out_shape=jax.ShapeDtypeStruct((G, 1, H), F32),
    )(pts, pos, posT, *args)

    xp = xp.reshape(B, S, H)
    ep = p['enc']
    eargs = [ep['Wk'], _row(ep['bk']), ep['Wq'], _row(ep['bq']),
             ep['Wv'], _row(ep['bv']), ep['Wo1'], _row(ep['bo1']),
             ep['Wo2'], _row(ep['bo2']),
             _row(ep['ln_scale']), _row(ep['ln_bias'])]
    out = pl.pallas_call(
        _enc_kernel,
        in_specs=[pl.BlockSpec(xp.shape, lambda: (0, 0, 0))] +
                 [pl.BlockSpec(a.shape, lambda: (0,) * a.ndim) for a in eargs],
        out_specs=pl.BlockSpec((B, H), lambda: (0, 0)),
        out_shape=jax.ShapeDtypeStruct((B, H), F32),
    )(xp, *eargs)
    return out
